# Initial kernel scaffold; baseline (speedup 1.0000x reference)
import jax
import jax.numpy as jnp
from jax.experimental import pallas as pl


def kernel(pred, target):
    raise NotImplementedError("write your pallas kernel here")


# speedup vs baseline: 12.2212x; 12.2212x over previous
import functools

import jax
import jax.numpy as jnp
from jax import lax
from jax.experimental import pallas as pl
from jax.experimental.pallas import tpu as pltpu
from jax.experimental.pallas import tpu_sc as plsc

NB = 2048
NB2 = 2 * NB
HP = NB2 + 1
K = 8192
NL = 16


def _sc_lovasz(pred_r, tgt_r):
    B, N = pred_r.shape
    rows_per_sc = B // 2
    tiles_per_row = 16 // rows_per_sc
    seg_len = N // tiles_per_row
    nchunks = seg_len // K
    mesh = plsc.VectorSubcoreMesh(core_axis_name="c", subcore_axis_name="s")

    @functools.partial(
        pl.kernel,
        mesh=mesh,
        out_type=jax.ShapeDtypeStruct((B, NL), jnp.float32),
        compiler_params=pltpu.CompilerParams(needs_layout_passes=False),
        scratch_types=[
            pltpu.VMEM((K,), jnp.float32),
            pltpu.VMEM((K,), jnp.float32),
            pltpu.VMEM((NL, HP), jnp.float32),
            pltpu.VMEM((NB2,), jnp.float32),
            pltpu.VMEM((NB2,), jnp.float32),
            pltpu.VMEM((2, NL), jnp.float32),
            pltpu.VMEM_SHARED((16, 2, NL), jnp.float32),
            pltpu.VMEM_SHARED((16, NB2), jnp.float32),
        ],
    )
    def body(pred_hbm, tgt_hbm, out_hbm, pbuf, tbuf, hist, hred, hcomb, svec,
             stats, hists):
        c = lax.axis_index("c")
        s = lax.axis_index("s")
        row = c * rows_per_sc + s // tiles_per_row
        seg = s % tiles_per_row
        base = seg * seg_len
        ones = jnp.ones((NL,), jnp.float32)
        zeros = jnp.zeros((NL,), jnp.float32)
        lane = lax.broadcasted_iota(jnp.int32, (NL,), 0)

        def zero_body(k, _):
            for l in range(NL):
                hist[l, pl.ds(k * NL, NL)] = zeros
            return 0
        lax.fori_loop(0, NB2 // NL, zero_body, 0)

        def pa_chunk(ci, carry):
            macc, tacc = carry
            pltpu.sync_copy(pred_hbm.at[row, pl.ds(base + ci * K, K)], pbuf)
            pltpu.sync_copy(tgt_hbm.at[row, pl.ds(base + ci * K, K)], tbuf)

            def pa_step(i, carry2):
                m2, t2 = carry2
                pv = pbuf[pl.ds(i * NL, NL)]
                tv = tbuf[pl.ds(i * NL, NL)]
                e = 1.0 - pv * (2.0 * tv - 1.0)
                return jnp.maximum(m2, e), t2 + tv
            return lax.fori_loop(0, K // NL, pa_step, (macc, tacc))

        macc, tacc = lax.fori_loop(
            0, nchunks, pa_chunk,
            (jnp.full((NL,), -jnp.inf, jnp.float32), zeros))

        svec[0, :] = macc
        svec[1, :] = tacc
        pltpu.sync_copy(svec, stats.at[s])
        plsc.subcore_barrier()
        r0 = (s // tiles_per_row) * tiles_per_row
        mvec = jnp.full((NL,), -jnp.inf, jnp.float32)
        tvec = zeros
        for j in range(tiles_per_row):
            pltpu.sync_copy(stats.at[r0 + j], svec)
            mvec = jnp.maximum(mvec, svec[0, :])
            tvec = tvec + svec[1, :]
        m = jnp.maximum(jnp.max(mvec), 0.0)
        ptot = jnp.sum(tvec)
        fnb = jnp.float32(NB)
        scale = (jnp.full((NL,), fnb) /
                 jnp.maximum(jnp.full((NL,), m), jnp.float32(1e-30)))

        def pb_chunk(ci, _):
            pltpu.sync_copy(pred_hbm.at[row, pl.ds(base + ci * K, K)], pbuf)
            pltpu.sync_copy(tgt_hbm.at[row, pl.ds(base + ci * K, K)], tbuf)

            def pb_step(i, _2):
                pv = pbuf[pl.ds(i * NL, NL)]
                tv = tbuf[pl.ds(i * NL, NL)]
                e = 1.0 - pv * (2.0 * tv - 1.0)
                msk = e > 0.0
                af = jnp.minimum(e * scale, fnb)
                lin = jnp.minimum(af.astype(jnp.int32), NB - 1)
                lin = jnp.maximum(lin, 0)
                idx2 = (NB - 1) - lin + tv.astype(jnp.int32) * NB
                plsc.addupdate_scatter(hist, [lane, idx2], ones, mask=msk)
                return 0
            lax.fori_loop(0, K // NL, pb_step, 0)
            return 0
        lax.fori_loop(0, nchunks, pb_chunk, 0)

        def lr_body(k, _):
            acc = zeros
            for l in range(NL):
                acc = acc + hist[l, pl.ds(k * NL, NL)]
            hred[pl.ds(k * NL, NL)] = acc
            return 0
        lax.fori_loop(0, NB2 // NL, lr_body, 0)
        pltpu.sync_copy(hred, hists.at[s])
        plsc.subcore_barrier()

        @pl.when(seg == 0)
        def _():
            pltpu.sync_copy(hists.at[r0], hcomb)
            for j in range(1, tiles_per_row):
                pltpu.sync_copy(hists.at[r0 + j], hred)

                def add_body(k, _):
                    sl = pl.ds(k * NL, NL)
                    hcomb[sl] = hcomb[sl] + hred[sl]
                    return 0
                lax.fori_loop(0, NB2 // NL, add_body, 0)

            w = m * jnp.float32(1.0 / NB)

            def scan_body(k, carry):
                acc, rcar, pcar = carry
                negv = hcomb[pl.ds(k * NL, NL)]
                posv = hcomb[pl.ds(NB + k * NL, NL)]
                nv = negv + posv
                cn = jnp.cumsum(nv) + rcar
                cp = jnp.cumsum(posv) + pcar
                rb = cn - nv
                pb = cp - posv
                ga = jnp.where(cn == 0.0, 0.0,
                               1.0 - (ptot - cp) / (ptot + cn - cp))
                gb = jnp.where(rb == 0.0, 0.0,
                               1.0 - (ptot - pb) / (ptot + rb - pb))
                dvec = (jnp.float32(k * NL) +
                        lax.broadcasted_iota(jnp.int32, (NL,), 0)
                        .astype(jnp.float32))
                mid = (fnb - 0.5 - dvec) * w
                acc = acc + mid * (ga - gb)
                return acc, rcar + jnp.sum(nv), pcar + jnp.sum(posv)

            acc, _r, _p = lax.fori_loop(
                0, NB // NL, scan_body,
                (zeros, jnp.float32(0.0), jnp.float32(0.0)))
            loss = jnp.sum(acc)
            svec[0, :] = jnp.full((NL,), loss)
            pltpu.sync_copy(svec.at[0], out_hbm.at[row])

    return body(pred_r, tgt_r)


def kernel(pred, target):
    B = pred.shape[0]
    pred_r = pred.reshape(B, -1)
    tgt_r = target.reshape(B, -1).astype(jnp.float32)
    out = _sc_lovasz(pred_r, tgt_r)
    return jnp.mean(out[:, 0])


# speedup vs baseline: 14.9441x; 1.2228x over previous
import functools

import jax
import jax.numpy as jnp
from jax import lax
from jax.experimental import pallas as pl
from jax.experimental.pallas import tpu as pltpu
from jax.experimental.pallas import tpu_sc as plsc

NB = 2048
NB2 = 2 * NB
K = 8192
NL = 16


def _sc_lovasz(pred_r, tgt_r):
    B, N = pred_r.shape
    rows_per_sc = B // 2
    tiles_per_row = 16 // rows_per_sc
    seg_len = N // tiles_per_row
    nchunks = seg_len // K
    mesh = plsc.VectorSubcoreMesh(core_axis_name="c", subcore_axis_name="s")

    @functools.partial(
        pl.kernel,
        mesh=mesh,
        out_type=jax.ShapeDtypeStruct((B, NL), jnp.float32),
        compiler_params=pltpu.CompilerParams(needs_layout_passes=False),
        scratch_types=[
            pltpu.VMEM((seg_len,), jnp.int32),
            pltpu.VMEM((2, K), jnp.float32),
            pltpu.VMEM((2, K), jnp.int32),
            pltpu.VMEM((NB2,), jnp.float32),
            pltpu.VMEM((NB2,), jnp.float32),
            pltpu.VMEM((NB2,), jnp.float32),
            pltpu.VMEM((2, NL), jnp.float32),
            pltpu.VMEM_SHARED((16, 2, NL), jnp.float32),
            pltpu.VMEM_SHARED((16, NB2), jnp.float32),
            pltpu.SemaphoreType.DMA,
            pltpu.SemaphoreType.DMA,
        ],
    )
    def body(pred_hbm, tgt_hbm, out_hbm, kbuf, pstage, tstage, hist, hred,
             hcomb, svec, stats, hists, sem0, sem1):
        c = lax.axis_index("c")
        s = lax.axis_index("s")
        row = c * rows_per_sc + s // tiles_per_row
        seg = s % tiles_per_row
        base = seg * seg_len
        ones = jnp.ones((NL,), jnp.float32)
        zeros = jnp.zeros((NL,), jnp.float32)
        fnb = jnp.float32(NB)
        sems = (sem0, sem1)

        def zero_body(k, _):
            hist[pl.ds(k * NL, NL)] = zeros
            return 0
        lax.fori_loop(0, NB2 // NL, zero_body, 0)

        def start(ci):
            b = ci % 2
            sl = pl.ds(base + ci * K, K)
            return (pltpu.async_copy(pred_hbm.at[row, sl], pstage.at[b], sems[b]),
                    pltpu.async_copy(tgt_hbm.at[row, sl], tstage.at[b], sems[b]))

        inflight = start(0)
        macc = jnp.full((NL,), -jnp.inf, jnp.float32)
        tacc = zeros
        for ci in range(nchunks):
            nxt = start(ci + 1) if ci + 1 < nchunks else None
            inflight[0].wait()
            inflight[1].wait()
            b = ci % 2

            def pa_step(i, carry, _b=b, _ci=ci):
                m2, t2 = carry
                pv = pstage[_b, pl.ds(i * NL, NL)]
                tv = tstage[_b, pl.ds(i * NL, NL)]
                tf = tv.astype(jnp.float32)
                e = 1.0 - pv * (2.0 * tf - 1.0)
                a = jnp.maximum(e, 0.0)
                k = jnp.where(
                    e > 0.0,
                    (plsc.bitcast(a, jnp.int32) & jnp.int32(~1)) | tv,
                    jnp.zeros((NL,), jnp.int32))
                kbuf[pl.ds(_ci * K + i * NL, NL)] = k
                return jnp.maximum(m2, a), t2 + tf

            macc, tacc = lax.fori_loop(0, K // NL, pa_step, (macc, tacc))
            inflight = nxt

        svec[0, :] = macc
        svec[1, :] = tacc
        pltpu.sync_copy(svec, stats.at[s])
        plsc.subcore_barrier()
        r0 = (s // tiles_per_row) * tiles_per_row
        mvec = jnp.full((NL,), -jnp.inf, jnp.float32)
        tvec = zeros
        for j in range(tiles_per_row):
            pltpu.sync_copy(stats.at[r0 + j], svec)
            mvec = jnp.maximum(mvec, svec[0, :])
            tvec = tvec + svec[1, :]
        m = jnp.maximum(jnp.max(mvec), 0.0)
        ptot = jnp.sum(tvec)
        scale = (jnp.full((NL,), fnb) /
                 jnp.maximum(jnp.full((NL,), m), jnp.float32(1e-30)))

        def pb_step(i, _):
            k = kbuf[pl.ds(i * NL, NL)]
            msk = k > 1
            a = plsc.bitcast(k, jnp.float32)
            af = jnp.minimum(a * scale, fnb)
            lin = jnp.minimum(af.astype(jnp.int32), NB - 1)
            lin = jnp.maximum(lin, 0)
            idx2 = (NB - 1) - lin + (k & 1) * NB
            plsc.addupdate_scatter(hist, [idx2], ones, mask=msk)
            return 0
        lax.fori_loop(0, seg_len // NL, pb_step, 0)

        pltpu.sync_copy(hist, hists.at[s])
        plsc.subcore_barrier()

        @pl.when(seg == 0)
        def _():
            pltpu.sync_copy(hists.at[r0], hcomb)
            for j in range(1, tiles_per_row):
                pltpu.sync_copy(hists.at[r0 + j], hred)

                def add_body(k, _):
                    sl = pl.ds(k * NL, NL)
                    hcomb[sl] = hcomb[sl] + hred[sl]
                    return 0
                lax.fori_loop(0, NB2 // NL, add_body, 0)

            w = m * jnp.float32(1.0 / NB)

            def scan_body(k, carry):
                acc, rcar, pcar = carry
                negv = hcomb[pl.ds(k * NL, NL)]
                posv = hcomb[pl.ds(NB + k * NL, NL)]
                nv = negv + posv
                cn = jnp.cumsum(nv) + rcar
                cp = jnp.cumsum(posv) + pcar
                rb = cn - nv
                pb = cp - posv
                ga = jnp.where(cn == 0.0, 0.0,
                               1.0 - (ptot - cp) / (ptot + cn - cp))
                gb = jnp.where(rb == 0.0, 0.0,
                               1.0 - (ptot - pb) / (ptot + rb - pb))
                dvec = (jnp.float32(k * NL) +
                        lax.broadcasted_iota(jnp.int32, (NL,), 0)
                        .astype(jnp.float32))
                mid = (fnb - 0.5 - dvec) * w
                acc = acc + mid * (ga - gb)
                return acc, rcar + jnp.sum(nv), pcar + jnp.sum(posv)

            acc, _r, _p = lax.fori_loop(
                0, NB // NL, scan_body,
                (zeros, jnp.float32(0.0), jnp.float32(0.0)))
            loss = jnp.sum(acc)
            svec[0, :] = jnp.full((NL,), loss)
            pltpu.sync_copy(svec.at[0], out_hbm.at[row])

    return body(pred_r, tgt_r)


def kernel(pred, target):
    B = pred.shape[0]
    pred_r = pred.reshape(B, -1)
    tgt_r = target.reshape(B, -1)
    out = _sc_lovasz(pred_r, tgt_r)
    return jnp.mean(out[:, 0])


# speedup vs baseline: 15.1323x; 1.0126x over previous
import functools

import jax
import jax.numpy as jnp
from jax import lax
from jax.experimental import pallas as pl
from jax.experimental.pallas import tpu as pltpu
from jax.experimental.pallas import tpu_sc as plsc

NB = 2048
NB2 = 2 * NB
K = 8192
NL = 16


def _sc_lovasz(pred_r, tgt_r):
    B, N = pred_r.shape
    rows_per_sc = B // 2
    tiles_per_row = 16 // rows_per_sc
    seg_len = N // tiles_per_row
    nchunks = seg_len // K
    mesh = plsc.VectorSubcoreMesh(core_axis_name="c", subcore_axis_name="s")

    @functools.partial(
        pl.kernel,
        mesh=mesh,
        out_type=jax.ShapeDtypeStruct((B, NL), jnp.float32),
        compiler_params=pltpu.CompilerParams(needs_layout_passes=False),
        scratch_types=[
            pltpu.VMEM((seg_len,), jnp.int32),
            pltpu.VMEM((2, K), jnp.float32),
            pltpu.VMEM((2, K), jnp.int32),
            pltpu.VMEM((NB2,), jnp.float32),
            pltpu.VMEM((NB2,), jnp.float32),
            pltpu.VMEM((NB2,), jnp.float32),
            pltpu.VMEM((2, NL), jnp.float32),
            pltpu.VMEM_SHARED((16, 2, NL), jnp.float32),
            pltpu.VMEM_SHARED((16, NB2), jnp.float32),
            pltpu.SemaphoreType.DMA,
            pltpu.SemaphoreType.DMA,
        ],
    )
    def body(pred_hbm, tgt_hbm, out_hbm, kbuf, pstage, tstage, hist, hred,
             hcomb, svec, stats, hists, sem0, sem1):
        c = lax.axis_index("c")
        s = lax.axis_index("s")
        row = c * rows_per_sc + s // tiles_per_row
        seg = s % tiles_per_row
        base = seg * seg_len
        ones = jnp.ones((NL,), jnp.float32)
        zeros = jnp.zeros((NL,), jnp.float32)
        fnb = jnp.float32(NB)
        sems = (sem0, sem1)

        def zero_body(k, _):
            for u in range(4):
                hist[pl.ds(k * (4 * NL) + u * NL, NL)] = zeros
            return 0
        lax.fori_loop(0, NB2 // (4 * NL), zero_body, 0)

        def start(ci):
            b = ci % 2
            sl = pl.ds(base + ci * K, K)
            return (pltpu.async_copy(pred_hbm.at[row, sl], pstage.at[b], sems[b]),
                    pltpu.async_copy(tgt_hbm.at[row, sl], tstage.at[b], sems[b]))

        inflight = start(0)
        neginf = jnp.full((NL,), -jnp.inf, jnp.float32)
        accs = tuple(x for _ in range(4) for x in (neginf, zeros))
        for ci in range(nchunks):
            nxt = start(ci + 1) if ci + 1 < nchunks else None
            inflight[0].wait()
            inflight[1].wait()
            b = ci % 2

            def pa_step(i, carry, _b=b, _ci=ci):
                out = []
                for u in range(4):
                    off = pl.ds(i * (4 * NL) + u * NL, NL)
                    pv = pstage[_b, off]
                    tv = tstage[_b, off]
                    tf = tv.astype(jnp.float32)
                    e = 1.0 - pv * (2.0 * tf - 1.0)
                    a = jnp.maximum(e, 0.0)
                    k = jnp.where(
                        e > 0.0,
                        (plsc.bitcast(a, jnp.int32) & jnp.int32(~1)) | tv,
                        jnp.zeros((NL,), jnp.int32))
                    kbuf[pl.ds(_ci * K + i * (4 * NL) + u * NL, NL)] = k
                    out.extend((jnp.maximum(carry[2 * u], a),
                                carry[2 * u + 1] + tf))
                return tuple(out)

            accs = lax.fori_loop(0, K // (4 * NL), pa_step, accs)
            inflight = nxt
        macc = jnp.maximum(jnp.maximum(accs[0], accs[2]),
                           jnp.maximum(accs[4], accs[6]))
        tacc = (accs[1] + accs[3]) + (accs[5] + accs[7])

        svec[0, :] = macc
        svec[1, :] = tacc
        pltpu.sync_copy(svec, stats.at[s])
        plsc.subcore_barrier()
        r0 = (s // tiles_per_row) * tiles_per_row
        mvec = jnp.full((NL,), -jnp.inf, jnp.float32)
        tvec = zeros
        for j in range(tiles_per_row):
            pltpu.sync_copy(stats.at[r0 + j], svec)
            mvec = jnp.maximum(mvec, svec[0, :])
            tvec = tvec + svec[1, :]
        m = jnp.maximum(jnp.max(mvec), 0.0)
        ptot = jnp.sum(tvec)
        scale = (jnp.full((NL,), fnb) /
                 jnp.maximum(jnp.full((NL,), m), jnp.float32(1e-30)))

        def pb_step(i, _):
            for u in range(4):
                k = kbuf[pl.ds(i * (4 * NL) + u * NL, NL)]
                msk = k > 1
                a = plsc.bitcast(k, jnp.float32)
                af = jnp.minimum(a * scale, fnb)
                lin = jnp.minimum(af.astype(jnp.int32), NB - 1)
                lin = jnp.maximum(lin, 0)
                idx2 = (NB - 1) - lin + (k & 1) * NB
                plsc.addupdate_scatter(hist, [idx2], ones, mask=msk)
            return 0
        lax.fori_loop(0, seg_len // (4 * NL), pb_step, 0)

        pltpu.sync_copy(hist, hists.at[s])
        plsc.subcore_barrier()

        @pl.when(seg == 0)
        def _():
            pltpu.sync_copy(hists.at[r0], hcomb)
            for j in range(1, tiles_per_row):
                pltpu.sync_copy(hists.at[r0 + j], hred)

                def add_body(k, _):
                    for u in range(4):
                        sl = pl.ds(k * (4 * NL) + u * NL, NL)
                        hcomb[sl] = hcomb[sl] + hred[sl]
                    return 0
                lax.fori_loop(0, NB2 // (4 * NL), add_body, 0)

            w = m * jnp.float32(1.0 / NB)

            def scan_body(k, carry):
                acc, rcar, pcar = carry
                negv = hcomb[pl.ds(k * NL, NL)]
                posv = hcomb[pl.ds(NB + k * NL, NL)]
                nv = negv + posv
                cn = jnp.cumsum(nv) + rcar
                cp = jnp.cumsum(posv) + pcar
                rb = cn - nv
                pb = cp - posv
                ga = jnp.where(cn == 0.0, 0.0,
                               1.0 - (ptot - cp) / (ptot + cn - cp))
                gb = jnp.where(rb == 0.0, 0.0,
                               1.0 - (ptot - pb) / (ptot + rb - pb))
                dvec = (jnp.float32(k * NL) +
                        lax.broadcasted_iota(jnp.int32, (NL,), 0)
                        .astype(jnp.float32))
                mid = (fnb - 0.5 - dvec) * w
                acc = acc + mid * (ga - gb)
                return acc, rcar + jnp.sum(nv), pcar + jnp.sum(posv)

            acc, _r, _p = lax.fori_loop(
                0, NB // NL, scan_body,
                (zeros, jnp.float32(0.0), jnp.float32(0.0)))
            loss = jnp.sum(acc)
            svec[0, :] = jnp.full((NL,), loss)
            pltpu.sync_copy(svec.at[0], out_hbm.at[row])

    return body(pred_r, tgt_r)


def kernel(pred, target):
    B = pred.shape[0]
    pred_r = pred.reshape(B, -1)
    tgt_r = target.reshape(B, -1)
    out = _sc_lovasz(pred_r, tgt_r)
    return jnp.mean(out[:, 0])


# speedup vs baseline: 20.2080x; 1.3354x over previous
import functools

import jax
import jax.numpy as jnp
from jax import lax
from jax.experimental import pallas as pl
from jax.experimental.pallas import tpu as pltpu
from jax.experimental.pallas import tpu_sc as plsc

NB = 8192
NB2 = 2 * NB
BASE = 93 * 128
K = 16384
NL = 16
NQ = NB // 4


def _sc_lovasz(pred_r, tgt_r):
    B, N = pred_r.shape
    rows_per_sc = B // 2
    tiles_per_row = 16 // rows_per_sc
    seg_len = N // tiles_per_row
    nchunks = seg_len // K
    mesh = plsc.VectorSubcoreMesh(core_axis_name="c", subcore_axis_name="s")

    @functools.partial(
        pl.kernel,
        mesh=mesh,
        out_type=(jax.ShapeDtypeStruct((B, NL), jnp.float32),
                  jax.ShapeDtypeStruct((2, 16, NB2), jnp.float32)),
        compiler_params=pltpu.CompilerParams(needs_layout_passes=False),
        scratch_types=[
            pltpu.VMEM((2, K), jnp.float32),
            pltpu.VMEM((2, K), jnp.int32),
            pltpu.VMEM((NB2,), jnp.float32),
            pltpu.VMEM((NQ,), jnp.float32),
            pltpu.VMEM((NQ,), jnp.float32),
            pltpu.VMEM((6, NQ), jnp.float32),
            pltpu.VMEM((2, NL), jnp.float32),
            pltpu.SMEM((4,), jnp.int32),
            pltpu.SemaphoreType.DMA,
            pltpu.SemaphoreType.DMA,
        ],
    )
    def body(pred_hbm, tgt_hbm, out_hbm, hx_hbm, pstage, tstage, hist, qn, qp,
             rdb, svec, cnt, sem0, sem1):
        c = lax.axis_index("c")
        s = lax.axis_index("s")
        row = c * rows_per_sc + s // tiles_per_row
        seg = s % tiles_per_row
        base = seg * seg_len
        ones = jnp.ones((NL,), jnp.float32)
        zeros = jnp.zeros((NL,), jnp.float32)
        izeros = jnp.zeros((NL,), jnp.int32)
        sems = (sem0, sem1)

        cnt[0] = 0
        cnt[1] = 0
        cnt[2] = 0
        cnt[3] = 0

        def zero_body(k, _):
            for u in range(4):
                hist[pl.ds(k * (4 * NL) + u * NL, NL)] = zeros
            return 0
        lax.fori_loop(0, NB2 // (4 * NL), zero_body, 0)

        def start(ci):
            b = ci % 2
            sl = pl.ds(base + ci * K, K)
            return (pltpu.async_copy(pred_hbm.at[row, sl], pstage.at[b], sems[b]),
                    pltpu.async_copy(tgt_hbm.at[row, sl], tstage.at[b], sems[b]))

        inflight = start(0)
        taccs = [izeros] * 4
        for ci in range(nchunks):
            nxt = start(ci + 1) if ci + 1 < nchunks else None
            inflight[0].wait()
            inflight[1].wait()
            b = ci % 2

            def st_step(i, carry, _b=b):
                out = []
                for u in range(4):
                    off = pl.ds(i * (4 * NL) + u * NL, NL)
                    pv = pstage[_b, off]
                    tv = tstage[_b, off]
                    sign = plsc.bitcast(
                        (tv << 31) ^ jnp.int32(-1082130432), jnp.float32)
                    e = 1.0 - pv * sign
                    a = jnp.maximum(e, 0.0)
                    u16 = lax.shift_right_logical(
                        plsc.bitcast(a, jnp.int32), 16)
                    lin = jnp.minimum(jnp.maximum(u16 - BASE, 0), NB - 1)
                    idx2 = (NB - 1) - lin + (tv << 13)
                    plsc.addupdate_scatter(hist, [idx2], ones, mask=e > 0.0)
                    out.append(carry[u] + tv)
                return tuple(out)

            taccs = lax.fori_loop(0, K // (4 * NL), st_step, tuple(taccs))
            inflight = nxt
        tsum = jnp.sum((taccs[0] + taccs[1]) + (taccs[2] + taccs[3]))

        pltpu.sync_copy(hist, hx_hbm.at[c, s])
        plsc.subcore_barrier()

        r0 = (s // tiles_per_row) * tiles_per_row
        for j in range(tiles_per_row):
            plsc.fetch_and_add(cnt.at[0], tsum, subcore_id=r0 + j)

        qoff = seg * NQ
        pltpu.sync_copy(hx_hbm.at[c, r0, pl.ds(qoff, NQ)], qn)
        pltpu.sync_copy(hx_hbm.at[c, r0, pl.ds(NB + qoff, NQ)], qp)
        for j in range(1, tiles_per_row):
            pltpu.sync_copy(hx_hbm.at[c, r0 + j, pl.ds(qoff, NQ)],
                            rdb.at[j - 1])
            pltpu.sync_copy(hx_hbm.at[c, r0 + j, pl.ds(NB + qoff, NQ)],
                            rdb.at[2 + j])

        def addall(k, _):
            for u in range(4):
                sl = pl.ds(k * (4 * NL) + u * NL, NL)
                pv = ((qp[sl] + rdb[3, sl]) + (rdb[4, sl] + rdb[5, sl]))
                nv = ((qn[sl] + rdb[0, sl]) + (rdb[1, sl] + rdb[2, sl])) + pv
                qp[sl] = pv
                qn[sl] = nv
            return 0
        lax.fori_loop(0, NQ // (4 * NL), addall, 0)

        def qtot(k, carry):
            av, bv = carry
            return (av + qn[pl.ds(k * NL, NL)], bv + qp[pl.ds(k * NL, NL)])
        qnv, qpv = lax.fori_loop(0, NQ // NL, qtot, (zeros, zeros))
        qni = jnp.sum(qnv.astype(jnp.int32))
        qpi = jnp.sum(qpv.astype(jnp.int32))
        for j in range(1, tiles_per_row):
            @pl.when(seg < jnp.int32(j))
            def _(j=j):
                plsc.fetch_and_add(cnt.at[1], qni, subcore_id=r0 + j)
                plsc.fetch_and_add(cnt.at[2], qpi, subcore_id=r0 + j)
        plsc.subcore_barrier()

        ptotv = jnp.full((NL,), cnt[0]).astype(jnp.float32)
        rcar = jnp.full((NL,), cnt[1]).astype(jnp.float32)
        pcar = jnp.full((NL,), cnt[2]).astype(jnp.float32)

        def scan_body(k, carry):
            acc, rc, pc = carry
            nv = qn[pl.ds(k * NL, NL)]
            posv = qp[pl.ds(k * NL, NL)]
            cn = jnp.cumsum(nv) + rc
            cp = jnp.cumsum(posv) + pc
            rb = cn - nv
            pb = cp - posv
            ga = jnp.where(cn == 0.0, 0.0,
                           1.0 - (ptotv - cp) / (ptotv + cn - cp))
            gb = jnp.where(rb == 0.0, 0.0,
                           1.0 - (ptotv - pb) / (ptotv + rb - pb))
            dvec = (jnp.int32(qoff) + jnp.int32(k * NL) +
                    lax.broadcasted_iota(jnp.int32, (NL,), 0))
            kk = (BASE + (NB - 1)) - dvec
            mid = plsc.bitcast((kk << 16) | jnp.int32(0x8000), jnp.float32)
            acc = acc + mid * (ga - gb)
            return acc, rc + jnp.sum(nv), pc + jnp.sum(posv)

        acc, _r, _p = lax.fori_loop(0, NQ // NL, scan_body,
                                    (zeros, rcar, pcar))
        pfx = jnp.sum((acc * jnp.float32(4194304.0)).astype(jnp.int32))
        plsc.fetch_and_add(cnt.at[3], pfx, subcore_id=r0)
        plsc.subcore_barrier()

        @pl.when(seg == 0)
        def _():
            lsum = (jnp.full((NL,), cnt[3]).astype(jnp.float32) *
                    jnp.float32(1.0 / 4194304.0))
            svec[0, :] = lsum
            pltpu.sync_copy(svec.at[0], out_hbm.at[row])

    return body(pred_r, tgt_r)


def kernel(pred, target):
    B = pred.shape[0]
    pred_r = pred.reshape(B, -1)
    tgt_r = target.reshape(B, -1)
    out, _ = _sc_lovasz(pred_r, tgt_r)
    return jnp.mean(out[:, 0])
